# Initial kernel scaffold; baseline (speedup 1.0000x reference)
import jax
import jax.numpy as jnp
from jax.experimental import pallas as pl


def kernel(x, edge_index, batch, W1, b1, W2, b2, W3, b3, W4, b4, Wl, bl):
    raise NotImplementedError("write your pallas kernel here")


# speedup vs baseline: 11.2588x; 11.2588x over previous
import functools

import jax
import jax.numpy as jnp
from jax import lax
from jax.experimental import pallas as pl
from jax.experimental.pallas import tpu as pltpu
from jax.experimental.pallas import tpu_sc as plsc

NC = 2
NS = 16
NW = NC * NS
EK = 128

_mesh = plsc.VectorSubcoreMesh(
    core_axis_name="c", subcore_axis_name="s", num_cores=NC, num_subcores=NS)


def _edge_scatter(ce, m, d):
  rt = m // NS

  @functools.partial(
      pl.kernel,
      out_type=jax.ShapeDtypeStruct((NC, m, d), jnp.float32),
      mesh=_mesh,
      compiler_params=pltpu.CompilerParams(use_tc_tiling_on_sc=False),
      scratch_types=[
          pltpu.VMEM_SHARED((m, d), jnp.float32),
          pltpu.VMEM((EK,), jnp.int32),
          pltpu.VMEM((EK,), jnp.int32),
          pltpu.VMEM((EK, d), jnp.float32),
          pltpu.SemaphoreType.DMA,
      ],
  )
  def k(g_hbm, src_hbm, dst_hbm, zeros_hbm, out_hbm, acc, src_ix, dst_ix,
        msg, sem):
    c = lax.axis_index("c")
    s = lax.axis_index("s")
    wid = c * NS + s
    pltpu.sync_copy(zeros_hbm, acc.at[pl.ds(s * rt, rt)])
    plsc.subcore_barrier()

    def step(j, carry):
      base = (wid * ce + j) * EK
      pltpu.sync_copy(src_hbm.at[pl.ds(base, EK)], src_ix)
      pltpu.sync_copy(dst_hbm.at[pl.ds(base, EK)], dst_ix)
      pltpu.async_copy(g_hbm.at[src_ix], msg, sem).wait()
      pltpu.sync_copy(msg, acc.at[dst_ix], add=True)
      return carry

    lax.fori_loop(0, ce, step, 0)
    plsc.subcore_barrier()
    pltpu.sync_copy(acc.at[pl.ds(s * rt, rt)],
                    out_hbm.at[c, pl.ds(s * rt, rt)])

  return k


def _ones_scatter(ce, m):
  rt = m // NS

  @functools.partial(
      pl.kernel,
      out_type=jax.ShapeDtypeStruct((NC, m, 16), jnp.float32),
      mesh=_mesh,
      compiler_params=pltpu.CompilerParams(use_tc_tiling_on_sc=False),
      scratch_types=[
          pltpu.VMEM_SHARED((m, 16), jnp.float32),
          pltpu.VMEM((EK,), jnp.int32),
          pltpu.VMEM((EK, 16), jnp.float32),
      ],
  )
  def k(dst_hbm, ones_hbm, zeros_hbm, out_hbm, acc, dst_ix, ones_st):
    c = lax.axis_index("c")
    s = lax.axis_index("s")
    wid = c * NS + s
    pltpu.sync_copy(zeros_hbm, acc.at[pl.ds(s * rt, rt)])
    pltpu.sync_copy(ones_hbm, ones_st)
    plsc.subcore_barrier()

    def step(j, carry):
      base = (wid * ce + j) * EK
      pltpu.sync_copy(dst_hbm.at[pl.ds(base, EK)], dst_ix)
      pltpu.sync_copy(ones_st, acc.at[dst_ix], add=True)
      return carry

    lax.fori_loop(0, ce, step, 0)
    plsc.subcore_barrier()
    pltpu.sync_copy(acc.at[pl.ds(s * rt, rt)],
                    out_hbm.at[c, pl.ds(s * rt, rt)])

  return k


def _pool_scatter(cp, mp, d):
  rt = mp // NS

  @functools.partial(
      pl.kernel,
      out_type=(jax.ShapeDtypeStruct((NC, mp, d), jnp.float32),
                jax.ShapeDtypeStruct((NC, mp, 16), jnp.float32)),
      mesh=_mesh,
      compiler_params=pltpu.CompilerParams(use_tc_tiling_on_sc=False),
      scratch_types=[
          pltpu.VMEM_SHARED((mp, d), jnp.float32),
          pltpu.VMEM_SHARED((mp, 16), jnp.float32),
          pltpu.VMEM((EK,), jnp.int32),
          pltpu.VMEM((EK,), jnp.int32),
          pltpu.VMEM((EK, d), jnp.float32),
          pltpu.VMEM((EK, 16), jnp.float32),
          pltpu.SemaphoreType.DMA,
      ],
  )
  def k(h_hbm, nid_hbm, bat_hbm, ones_hbm, zs_hbm, zc_hbm, sums_hbm,
        cnts_hbm, acc_s, acc_c, nid_ix, bat_ix, msg, ones_st, sem):
    c = lax.axis_index("c")
    s = lax.axis_index("s")
    wid = c * NS + s
    pltpu.sync_copy(zs_hbm, acc_s.at[pl.ds(s * rt, rt)])
    pltpu.sync_copy(zc_hbm, acc_c.at[pl.ds(s * rt, rt)])
    pltpu.sync_copy(ones_hbm, ones_st)
    plsc.subcore_barrier()

    def step(j, carry):
      base = (wid * cp + j) * EK
      pltpu.sync_copy(nid_hbm.at[pl.ds(base, EK)], nid_ix)
      pltpu.sync_copy(bat_hbm.at[pl.ds(base, EK)], bat_ix)
      pltpu.async_copy(h_hbm.at[nid_ix], msg, sem).wait()
      pltpu.sync_copy(msg, acc_s.at[bat_ix], add=True)
      pltpu.sync_copy(ones_st, acc_c.at[bat_ix], add=True)
      return carry

    lax.fori_loop(0, cp, step, 0)
    plsc.subcore_barrier()
    pltpu.sync_copy(acc_s.at[pl.ds(s * rt, rt)],
                    sums_hbm.at[c, pl.ds(s * rt, rt)])
    pltpu.sync_copy(acc_c.at[pl.ds(s * rt, rt)],
                    cnts_hbm.at[c, pl.ds(s * rt, rt)])

  return k


def _tc_matmul(x, w):
  n, din = x.shape
  dout = w.shape[1]
  r = 1000

  def body(x_ref, w_ref, o_ref):
    o_ref[...] = jnp.dot(x_ref[...], w_ref[...],
                         preferred_element_type=jnp.float32)

  return pl.pallas_call(
      body,
      grid=(n // r,),
      in_specs=[pl.BlockSpec((r, din), lambda i: (i, 0)),
                pl.BlockSpec((din, dout), lambda i: (0, 0))],
      out_specs=pl.BlockSpec((r, dout), lambda i: (i, 0)),
      out_shape=jax.ShapeDtypeStruct((n, dout), jnp.float32),
  )(x, w)


def _tc_dinv_g(degp, h1):
  n, d1 = h1.shape
  r = 1000

  def body(dp_ref, h_ref, dv_ref, g_ref):
    deg = dp_ref[0, :, 0:1] + dp_ref[1, :, 0:1] + 1.0
    dv = lax.rsqrt(jnp.maximum(deg, 1e-12))
    dv_ref[...] = jnp.broadcast_to(dv, dv_ref.shape)
    g_ref[...] = dv * h_ref[...]

  return pl.pallas_call(
      body,
      grid=(n // r,),
      in_specs=[pl.BlockSpec((2, r, 16), lambda i: (0, i, 0)),
                pl.BlockSpec((r, d1), lambda i: (i, 0))],
      out_specs=[pl.BlockSpec((r, 8), lambda i: (i, 0)),
                 pl.BlockSpec((r, d1), lambda i: (i, 0))],
      out_shape=[jax.ShapeDtypeStruct((n, 8), jnp.float32),
                 jax.ShapeDtypeStruct((n, d1), jnp.float32)],
  )(degp, h1)


def _tc_layer(tp, h, dinv8, w, b2d):
  n, d = h.shape
  dn = w.shape[1]
  r = 1000

  def body(tp_ref, h_ref, dv_ref, w_ref, b_ref, hn_ref, gn_ref):
    dv = dv_ref[:, 0:1]
    t = tp_ref[0] + tp_ref[1]
    o = jnp.tanh(dv * t + (dv * dv) * h_ref[...] + b_ref[...])
    hn = jnp.dot(o, w_ref[...], preferred_element_type=jnp.float32)
    hn_ref[...] = hn
    gn_ref[...] = dv * hn

  return pl.pallas_call(
      body,
      grid=(n // r,),
      in_specs=[pl.BlockSpec((2, r, d), lambda i: (0, i, 0)),
                pl.BlockSpec((r, d), lambda i: (i, 0)),
                pl.BlockSpec((r, 8), lambda i: (i, 0)),
                pl.BlockSpec((d, dn), lambda i: (0, 0)),
                pl.BlockSpec((1, d), lambda i: (0, 0))],
      out_specs=[pl.BlockSpec((r, dn), lambda i: (i, 0)),
                 pl.BlockSpec((r, dn), lambda i: (i, 0))],
      out_shape=[jax.ShapeDtypeStruct((n, dn), jnp.float32),
                 jax.ShapeDtypeStruct((n, dn), jnp.float32)],
  )(tp, h, dinv8, w, b2d)


def _tc_last(tp, h, dinv8, b2d):
  n, d = h.shape
  r = 1000

  def body(tp_ref, h_ref, dv_ref, b_ref, o_ref):
    dv = dv_ref[:, 0:1]
    t = tp_ref[0] + tp_ref[1]
    o_ref[...] = dv * t + (dv * dv) * h_ref[...] + b_ref[...]

  return pl.pallas_call(
      body,
      grid=(n // r,),
      in_specs=[pl.BlockSpec((2, r, d), lambda i: (0, i, 0)),
                pl.BlockSpec((r, d), lambda i: (i, 0)),
                pl.BlockSpec((r, 8), lambda i: (i, 0)),
                pl.BlockSpec((1, d), lambda i: (0, 0))],
      out_specs=pl.BlockSpec((r, d), lambda i: (i, 0)),
      out_shape=jax.ShapeDtypeStruct((n, d), jnp.float32),
  )(tp, h, dinv8, b2d)


def _tc_final(sp, cp, wl, bl2d, g):
  mp, d = sp.shape[1], sp.shape[2]
  dn = wl.shape[1]

  def body(sp_ref, cp_ref, w_ref, b_ref, o_ref):
    sums = sp_ref[0, :g, :] + sp_ref[1, :g, :]
    cnt = cp_ref[0, :g, 0:1] + cp_ref[1, :g, 0:1]
    pooled = sums / jnp.maximum(cnt, 1.0)
    o_ref[...] = jnp.dot(pooled, w_ref[...],
                         preferred_element_type=jnp.float32) + b_ref[...]

  return pl.pallas_call(
      body,
      in_specs=[pl.BlockSpec((2, mp, d), lambda: (0, 0, 0)),
                pl.BlockSpec((2, mp, 16), lambda: (0, 0, 0)),
                pl.BlockSpec((d, dn), lambda: (0, 0)),
                pl.BlockSpec((1, dn), lambda: (0, 0))],
      out_specs=pl.BlockSpec((g, dn), lambda: (0, 0)),
      out_shape=jax.ShapeDtypeStruct((g, dn), jnp.float32),
  )(sp, cp, wl, bl2d)


def kernel(x, edge_index, batch, W1, b1, W2, b2, W3, b3, W4, b4, Wl, bl):
  n = x.shape[0]
  e = edge_index.shape[1]
  g = 64

  ce = -(-e // (NW * EK))
  e_pad = ce * NW * EK
  src_p = jnp.concatenate(
      [edge_index[0], jnp.zeros((e_pad - e,), jnp.int32)])
  dst_p = jnp.concatenate(
      [edge_index[1], jnp.full((e_pad - e,), n, jnp.int32)])
  m = -(-(n + 1) // (NS * 8)) * (NS * 8)

  cpool = -(-n // (NW * EK))
  n_pad = cpool * NW * EK
  nid_p = jnp.concatenate(
      [jnp.arange(n, dtype=jnp.int32), jnp.zeros((n_pad - n,), jnp.int32)])
  bat_p = jnp.concatenate(
      [batch, jnp.full((n_pad - n,), g, jnp.int32)])
  mp = -(-(g + 1) // (NS * 8)) * (NS * 8)

  ones16 = jnp.ones((EK, 16), jnp.float32)
  rt = m // NS
  rtp = mp // NS
  z16 = jnp.zeros((rt, 16), jnp.float32)
  zeros_d = {dd: jnp.zeros((rt, dd), jnp.float32) for dd in (16, 32, 64, 128)}
  zp128 = jnp.zeros((rtp, 128), jnp.float32)
  zp16 = jnp.zeros((rtp, 16), jnp.float32)

  degp = _ones_scatter(ce, m)(dst_p, ones16, z16)
  h1 = _tc_matmul(x, W1)
  dinv8, g1 = _tc_dinv_g(degp, h1)

  tp1 = _edge_scatter(ce, m, 16)(g1, src_p, dst_p, zeros_d[16])
  h2, g2 = _tc_layer(tp1, h1, dinv8, W2, b1[None, :])
  tp2 = _edge_scatter(ce, m, 32)(g2, src_p, dst_p, zeros_d[32])
  h3, g3 = _tc_layer(tp2, h2, dinv8, W3, b2[None, :])
  tp3 = _edge_scatter(ce, m, 64)(g3, src_p, dst_p, zeros_d[64])
  h4, g4 = _tc_layer(tp3, h3, dinv8, W4, b3[None, :])
  tp4 = _edge_scatter(ce, m, 128)(g4, src_p, dst_p, zeros_d[128])
  out4 = _tc_last(tp4, h4, dinv8, b4[None, :])

  sp, cp = _pool_scatter(cpool, mp, 128)(
      out4, nid_p, bat_p, ones16, zp128, zp16)
  return _tc_final(sp, cp, Wl, bl[None, :], g)


# speedup vs baseline: 11.4544x; 1.0174x over previous
import functools

import jax
import jax.numpy as jnp
from jax import lax
from jax.experimental import pallas as pl
from jax.experimental.pallas import tpu as pltpu
from jax.experimental.pallas import tpu_sc as plsc

NC = 2
NS = 16
NW = NC * NS
EK = 128

_mesh = plsc.VectorSubcoreMesh(
    core_axis_name="c", subcore_axis_name="s", num_cores=NC, num_subcores=NS)


def _edge_scatter(ce, ek, m, d):
  rt = m // NS

  @functools.partial(
      pl.kernel,
      out_type=jax.ShapeDtypeStruct((NC, m, d), jnp.float32),
      mesh=_mesh,
      compiler_params=pltpu.CompilerParams(use_tc_tiling_on_sc=False),
      scratch_types=[
          pltpu.VMEM_SHARED((m, d), jnp.float32),
          pltpu.VMEM((ce, ek), jnp.int32),
          pltpu.VMEM((ce, ek), jnp.int32),
          pltpu.VMEM((ek, d), jnp.float32),
          pltpu.VMEM((ek, d), jnp.float32),
          pltpu.SemaphoreType.DMA,
          pltpu.SemaphoreType.DMA,
      ],
  )
  def k(g_hbm, src_hbm, dst_hbm, zeros_hbm, out_hbm, acc, src_st, dst_st,
        msg0, msg1, sem0, sem1):
    c = lax.axis_index("c")
    s = lax.axis_index("s")
    wid = c * NS + s
    pltpu.sync_copy(src_hbm.at[wid], src_st)
    pltpu.sync_copy(dst_hbm.at[wid], dst_st)
    pltpu.sync_copy(zeros_hbm, acc.at[pl.ds(s * rt, rt)])
    pltpu.async_copy(g_hbm.at[src_st.at[0]], msg0, sem0)
    plsc.subcore_barrier()

    msgs = (msg0, msg1)
    sems = (sem0, sem1)

    def step(gidx, carry):
      for b in (0, 1):
        j = 2 * gidx + b
        nxt = j + 1

        @pl.when(nxt < ce)
        def _():
          pltpu.async_copy(g_hbm.at[src_st.at[nxt]], msgs[1 - b],
                           sems[1 - b])

        pltpu.make_async_copy(g_hbm.at[pl.ds(0, ek)], msgs[b],
                              sems[b]).wait()
        pltpu.sync_copy(msgs[b], acc.at[dst_st.at[j]], add=True)
      return carry

    lax.fori_loop(0, ce // 2, step, 0)
    plsc.subcore_barrier()
    pltpu.sync_copy(acc.at[pl.ds(s * rt, rt)],
                    out_hbm.at[c, pl.ds(s * rt, rt)])

  return k


def _ones_scatter(ce, m):
  rt = m // NS

  @functools.partial(
      pl.kernel,
      out_type=jax.ShapeDtypeStruct((NC, m, 16), jnp.float32),
      mesh=_mesh,
      compiler_params=pltpu.CompilerParams(use_tc_tiling_on_sc=False),
      scratch_types=[
          pltpu.VMEM_SHARED((m, 16), jnp.float32),
          pltpu.VMEM((ce, EK), jnp.int32),
          pltpu.VMEM((EK, 16), jnp.float32),
      ],
  )
  def k(dst_hbm, ones_hbm, zeros_hbm, out_hbm, acc, dst_st, ones_st):
    c = lax.axis_index("c")
    s = lax.axis_index("s")
    wid = c * NS + s
    pltpu.sync_copy(dst_hbm.at[wid], dst_st)
    pltpu.sync_copy(zeros_hbm, acc.at[pl.ds(s * rt, rt)])
    pltpu.sync_copy(ones_hbm, ones_st)
    plsc.subcore_barrier()

    def step(j, carry):
      pltpu.sync_copy(ones_st, acc.at[dst_st.at[j]], add=True)
      return carry

    lax.fori_loop(0, ce, step, 0)
    plsc.subcore_barrier()
    pltpu.sync_copy(acc.at[pl.ds(s * rt, rt)],
                    out_hbm.at[c, pl.ds(s * rt, rt)])

  return k


def _pool_scatter(cp, mp, d):
  rt = mp // NS

  @functools.partial(
      pl.kernel,
      out_type=(jax.ShapeDtypeStruct((NC, mp, d), jnp.float32),
                jax.ShapeDtypeStruct((NC, mp, 16), jnp.float32)),
      mesh=_mesh,
      compiler_params=pltpu.CompilerParams(use_tc_tiling_on_sc=False),
      scratch_types=[
          pltpu.VMEM_SHARED((mp, d), jnp.float32),
          pltpu.VMEM_SHARED((mp, 16), jnp.float32),
          pltpu.VMEM((cp, EK), jnp.int32),
          pltpu.VMEM((cp, EK), jnp.int32),
          pltpu.VMEM((EK, d), jnp.float32),
          pltpu.VMEM((EK, 16), jnp.float32),
          pltpu.SemaphoreType.DMA,
      ],
  )
  def k(h_hbm, nid_hbm, bat_hbm, ones_hbm, zs_hbm, zc_hbm, sums_hbm,
        cnts_hbm, acc_s, acc_c, nid_st, bat_st, msg, ones_st, sem):
    c = lax.axis_index("c")
    s = lax.axis_index("s")
    wid = c * NS + s
    pltpu.sync_copy(nid_hbm.at[wid], nid_st)
    pltpu.sync_copy(bat_hbm.at[wid], bat_st)
    pltpu.sync_copy(zs_hbm, acc_s.at[pl.ds(s * rt, rt)])
    pltpu.sync_copy(zc_hbm, acc_c.at[pl.ds(s * rt, rt)])
    pltpu.sync_copy(ones_hbm, ones_st)
    plsc.subcore_barrier()

    def step(j, carry):
      pltpu.async_copy(h_hbm.at[nid_st.at[j]], msg, sem).wait()
      pltpu.sync_copy(msg, acc_s.at[bat_st.at[j]], add=True)
      pltpu.sync_copy(ones_st, acc_c.at[bat_st.at[j]], add=True)
      return carry

    lax.fori_loop(0, cp, step, 0)
    plsc.subcore_barrier()
    pltpu.sync_copy(acc_s.at[pl.ds(s * rt, rt)],
                    sums_hbm.at[c, pl.ds(s * rt, rt)])
    pltpu.sync_copy(acc_c.at[pl.ds(s * rt, rt)],
                    cnts_hbm.at[c, pl.ds(s * rt, rt)])

  return k


def _tc_matmul(x, w):
  n, din = x.shape
  dout = w.shape[1]
  r = 1000

  def body(x_ref, w_ref, o_ref):
    o_ref[...] = jnp.dot(x_ref[...], w_ref[...],
                         preferred_element_type=jnp.float32)

  return pl.pallas_call(
      body,
      grid=(n // r,),
      in_specs=[pl.BlockSpec((r, din), lambda i: (i, 0)),
                pl.BlockSpec((din, dout), lambda i: (0, 0))],
      out_specs=pl.BlockSpec((r, dout), lambda i: (i, 0)),
      out_shape=jax.ShapeDtypeStruct((n, dout), jnp.float32),
  )(x, w)


def _tc_dinv_g(degp, h1):
  n, d1 = h1.shape
  r = 1000

  def body(dp_ref, h_ref, dv_ref, g_ref):
    deg = dp_ref[0, :, 0:1] + dp_ref[1, :, 0:1] + 1.0
    dv = lax.rsqrt(jnp.maximum(deg, 1e-12))
    dv_ref[...] = jnp.broadcast_to(dv, dv_ref.shape)
    g_ref[...] = dv * h_ref[...]

  return pl.pallas_call(
      body,
      grid=(n // r,),
      in_specs=[pl.BlockSpec((2, r, 16), lambda i: (0, i, 0)),
                pl.BlockSpec((r, d1), lambda i: (i, 0))],
      out_specs=[pl.BlockSpec((r, 8), lambda i: (i, 0)),
                 pl.BlockSpec((r, d1), lambda i: (i, 0))],
      out_shape=[jax.ShapeDtypeStruct((n, 8), jnp.float32),
                 jax.ShapeDtypeStruct((n, d1), jnp.float32)],
  )(degp, h1)


def _tc_layer(tp, h, dinv8, w, b2d):
  n, d = h.shape
  dn = w.shape[1]
  r = 1000

  def body(tp_ref, h_ref, dv_ref, w_ref, b_ref, hn_ref, gn_ref):
    dv = dv_ref[:, 0:1]
    t = tp_ref[0] + tp_ref[1]
    o = jnp.tanh(dv * t + (dv * dv) * h_ref[...] + b_ref[...])
    hn = jnp.dot(o, w_ref[...], preferred_element_type=jnp.float32)
    hn_ref[...] = hn
    gn_ref[...] = dv * hn

  return pl.pallas_call(
      body,
      grid=(n // r,),
      in_specs=[pl.BlockSpec((2, r, d), lambda i: (0, i, 0)),
                pl.BlockSpec((r, d), lambda i: (i, 0)),
                pl.BlockSpec((r, 8), lambda i: (i, 0)),
                pl.BlockSpec((d, dn), lambda i: (0, 0)),
                pl.BlockSpec((1, d), lambda i: (0, 0))],
      out_specs=[pl.BlockSpec((r, dn), lambda i: (i, 0)),
                 pl.BlockSpec((r, dn), lambda i: (i, 0))],
      out_shape=[jax.ShapeDtypeStruct((n, dn), jnp.float32),
                 jax.ShapeDtypeStruct((n, dn), jnp.float32)],
  )(tp, h, dinv8, w, b2d)


def _tc_last(tp, h, dinv8, b2d):
  n, d = h.shape
  r = 1000

  def body(tp_ref, h_ref, dv_ref, b_ref, o_ref):
    dv = dv_ref[:, 0:1]
    t = tp_ref[0] + tp_ref[1]
    o_ref[...] = dv * t + (dv * dv) * h_ref[...] + b_ref[...]

  return pl.pallas_call(
      body,
      grid=(n // r,),
      in_specs=[pl.BlockSpec((2, r, d), lambda i: (0, i, 0)),
                pl.BlockSpec((r, d), lambda i: (i, 0)),
                pl.BlockSpec((r, 8), lambda i: (i, 0)),
                pl.BlockSpec((1, d), lambda i: (0, 0))],
      out_specs=pl.BlockSpec((r, d), lambda i: (i, 0)),
      out_shape=jax.ShapeDtypeStruct((n, d), jnp.float32),
  )(tp, h, dinv8, b2d)


def _tc_final(sp, cp, wl, bl2d, g):
  mp, d = sp.shape[1], sp.shape[2]
  dn = wl.shape[1]

  def body(sp_ref, cp_ref, w_ref, b_ref, o_ref):
    sums = sp_ref[0, :g, :] + sp_ref[1, :g, :]
    cnt = cp_ref[0, :g, 0:1] + cp_ref[1, :g, 0:1]
    pooled = sums / jnp.maximum(cnt, 1.0)
    o_ref[...] = jnp.dot(pooled, w_ref[...],
                         preferred_element_type=jnp.float32) + b_ref[...]

  return pl.pallas_call(
      body,
      in_specs=[pl.BlockSpec((2, mp, d), lambda: (0, 0, 0)),
                pl.BlockSpec((2, mp, 16), lambda: (0, 0, 0)),
                pl.BlockSpec((d, dn), lambda: (0, 0)),
                pl.BlockSpec((1, dn), lambda: (0, 0))],
      out_specs=pl.BlockSpec((g, dn), lambda: (0, 0)),
      out_shape=jax.ShapeDtypeStruct((g, dn), jnp.float32),
  )(sp, cp, wl, bl2d)


def kernel(x, edge_index, batch, W1, b1, W2, b2, W3, b3, W4, b4, Wl, bl):
  n = x.shape[0]
  e = edge_index.shape[1]
  g = 64

  ce = -(-e // (NW * EK))
  ce = ce + (ce % 2)
  e_pad = ce * NW * EK
  src_flat = jnp.concatenate(
      [edge_index[0], jnp.zeros((e_pad - e,), jnp.int32)])
  dst_flat = jnp.concatenate(
      [edge_index[1], jnp.full((e_pad - e,), n, jnp.int32)])
  src_p = src_flat.reshape(NW, ce, EK)
  dst_p = dst_flat.reshape(NW, ce, EK)
  src_p64 = src_flat.reshape(NW, ce * 2, EK // 2)
  dst_p64 = dst_flat.reshape(NW, ce * 2, EK // 2)
  m = -(-(n + 1) // (NS * 8)) * (NS * 8)

  cpool = -(-n // (NW * EK))
  n_pad = cpool * NW * EK
  nid_p = jnp.concatenate(
      [jnp.arange(n, dtype=jnp.int32),
       jnp.zeros((n_pad - n,), jnp.int32)]).reshape(NW, cpool, EK)
  bat_p = jnp.concatenate(
      [batch, jnp.full((n_pad - n,), g, jnp.int32)]).reshape(NW, cpool, EK)
  mp = -(-(g + 1) // (NS * 8)) * (NS * 8)

  ones16 = jnp.ones((EK, 16), jnp.float32)
  rt = m // NS
  rtp = mp // NS
  z16 = jnp.zeros((rt, 16), jnp.float32)
  zeros_d = {dd: jnp.zeros((rt, dd), jnp.float32) for dd in (16, 32, 64, 128)}
  zp128 = jnp.zeros((rtp, 128), jnp.float32)
  zp16 = jnp.zeros((rtp, 16), jnp.float32)

  degp = _ones_scatter(ce, m)(dst_p, ones16, z16)
  h1 = _tc_matmul(x, W1)
  dinv8, g1 = _tc_dinv_g(degp, h1)

  tp1 = _edge_scatter(ce, EK, m, 16)(g1, src_p, dst_p, zeros_d[16])
  h2, g2 = _tc_layer(tp1, h1, dinv8, W2, b1[None, :])
  tp2 = _edge_scatter(ce, EK, m, 32)(g2, src_p, dst_p, zeros_d[32])
  h3, g3 = _tc_layer(tp2, h2, dinv8, W3, b2[None, :])
  tp3 = _edge_scatter(ce, EK, m, 64)(g3, src_p, dst_p, zeros_d[64])
  h4, g4 = _tc_layer(tp3, h3, dinv8, W4, b3[None, :])
  tp4 = _edge_scatter(ce * 2, EK // 2, m, 128)(g4, src_p64, dst_p64,
                                               zeros_d[128])
  out4 = _tc_last(tp4, h4, dinv8, b4[None, :])

  sp, cp = _pool_scatter(cpool, mp, 128)(
      out4, nid_p, bat_p, ones16, zp128, zp16)
  return _tc_final(sp, cp, Wl, bl[None, :], g)


# speedup vs baseline: 17.9191x; 1.5644x over previous
import functools

import jax
import jax.numpy as jnp
from jax import lax
from jax.experimental import pallas as pl
from jax.experimental.pallas import tpu as pltpu
from jax.experimental.pallas import tpu_sc as plsc

NC = 2
NS = 16
NW = NC * NS
EK = 128

_mesh = plsc.VectorSubcoreMesh(
    core_axis_name="c", subcore_axis_name="s", num_cores=NC, num_subcores=NS)


def _edge_scatter(ce, ek, m, d):
  rt = m // NS

  @functools.partial(
      pl.kernel,
      out_type=jax.ShapeDtypeStruct((NC, m, d), jnp.float32),
      mesh=_mesh,
      compiler_params=pltpu.CompilerParams(use_tc_tiling_on_sc=False),
      scratch_types=[
          pltpu.VMEM_SHARED((m, d), jnp.float32),
          pltpu.VMEM((ce, ek), jnp.int32),
          pltpu.VMEM((ce, ek), jnp.int32),
          pltpu.VMEM((ek, d), jnp.float32),
          pltpu.VMEM((ek, d), jnp.float32),
          pltpu.SemaphoreType.DMA,
          pltpu.SemaphoreType.DMA,
      ],
  )
  def k(g_hbm, src_hbm, dst_hbm, zeros_hbm, out_hbm, acc, src_st, dst_st,
        msg0, msg1, sem0, sem1):
    c = lax.axis_index("c")
    s = lax.axis_index("s")
    wid = c * NS + s
    pltpu.sync_copy(src_hbm.at[wid], src_st)
    pltpu.sync_copy(dst_hbm.at[wid], dst_st)
    pltpu.sync_copy(zeros_hbm, acc.at[pl.ds(s * rt, rt)])
    pltpu.async_copy(g_hbm.at[src_st.at[0]], msg0, sem0)
    plsc.subcore_barrier()

    msgs = (msg0, msg1)
    sems = (sem0, sem1)

    def step(gidx, carry):
      for b in (0, 1):
        j = 2 * gidx + b
        nxt = j + 1

        @pl.when(nxt < ce)
        def _():
          pltpu.async_copy(g_hbm.at[src_st.at[nxt]], msgs[1 - b],
                           sems[1 - b])

        pltpu.make_async_copy(g_hbm.at[pl.ds(0, ek)], msgs[b],
                              sems[b]).wait()
        pltpu.sync_copy(msgs[b], acc.at[dst_st.at[j]], add=True)
      return carry

    lax.fori_loop(0, ce // 2, step, 0)
    plsc.subcore_barrier()
    pltpu.sync_copy(acc.at[pl.ds(s * rt, rt)],
                    out_hbm.at[c, pl.ds(s * rt, rt)])

  return k


def _ones_scatter(ce, m):
  rt = m // NS

  @functools.partial(
      pl.kernel,
      out_type=jax.ShapeDtypeStruct((NC, m, 16), jnp.float32),
      mesh=_mesh,
      compiler_params=pltpu.CompilerParams(use_tc_tiling_on_sc=False),
      scratch_types=[
          pltpu.VMEM_SHARED((m, 16), jnp.float32),
          pltpu.VMEM((ce, EK), jnp.int32),
          pltpu.VMEM((EK, 16), jnp.float32),
      ],
  )
  def k(dst_hbm, ones_hbm, zeros_hbm, out_hbm, acc, dst_st, ones_st):
    c = lax.axis_index("c")
    s = lax.axis_index("s")
    wid = c * NS + s
    pltpu.sync_copy(dst_hbm.at[wid], dst_st)
    pltpu.sync_copy(zeros_hbm, acc.at[pl.ds(s * rt, rt)])
    pltpu.sync_copy(ones_hbm, ones_st)
    plsc.subcore_barrier()

    def step(j, carry):
      pltpu.sync_copy(ones_st, acc.at[dst_st.at[j]], add=True)
      return carry

    lax.fori_loop(0, ce, step, 0)
    plsc.subcore_barrier()
    pltpu.sync_copy(acc.at[pl.ds(s * rt, rt)],
                    out_hbm.at[c, pl.ds(s * rt, rt)])

  return k


def _pool_scatter(cp, mp, d):
  rt = mp // NS

  @functools.partial(
      pl.kernel,
      out_type=(jax.ShapeDtypeStruct((NC, mp, d), jnp.float32),
                jax.ShapeDtypeStruct((NC, mp, 16), jnp.float32)),
      mesh=_mesh,
      compiler_params=pltpu.CompilerParams(use_tc_tiling_on_sc=False),
      scratch_types=[
          pltpu.VMEM_SHARED((mp, d), jnp.float32),
          pltpu.VMEM_SHARED((mp, 16), jnp.float32),
          pltpu.VMEM((cp, EK), jnp.int32),
          pltpu.VMEM((cp, EK), jnp.int32),
          pltpu.VMEM((EK, d), jnp.float32),
          pltpu.VMEM((EK, 16), jnp.float32),
          pltpu.SemaphoreType.DMA,
      ],
  )
  def k(h_hbm, nid_hbm, bat_hbm, ones_hbm, zs_hbm, zc_hbm, sums_hbm,
        cnts_hbm, acc_s, acc_c, nid_st, bat_st, msg, ones_st, sem):
    c = lax.axis_index("c")
    s = lax.axis_index("s")
    wid = c * NS + s
    pltpu.sync_copy(nid_hbm.at[wid], nid_st)
    pltpu.sync_copy(bat_hbm.at[wid], bat_st)
    pltpu.sync_copy(zs_hbm, acc_s.at[pl.ds(s * rt, rt)])
    pltpu.sync_copy(zc_hbm, acc_c.at[pl.ds(s * rt, rt)])
    pltpu.sync_copy(ones_hbm, ones_st)
    plsc.subcore_barrier()

    def step(j, carry):
      pltpu.async_copy(h_hbm.at[nid_st.at[j]], msg, sem).wait()
      pltpu.sync_copy(msg, acc_s.at[bat_st.at[j]], add=True)
      pltpu.sync_copy(ones_st, acc_c.at[bat_st.at[j]], add=True)
      return carry

    lax.fori_loop(0, cp, step, 0)
    plsc.subcore_barrier()
    pltpu.sync_copy(acc_s.at[pl.ds(s * rt, rt)],
                    sums_hbm.at[c, pl.ds(s * rt, rt)])
    pltpu.sync_copy(acc_c.at[pl.ds(s * rt, rt)],
                    cnts_hbm.at[c, pl.ds(s * rt, rt)])

  return k


def _tc_matmul(x, w):
  n, din = x.shape
  dout = w.shape[1]
  r = 1000

  def body(x_ref, w_ref, o_ref):
    o_ref[...] = jnp.dot(x_ref[...], w_ref[...],
                         preferred_element_type=jnp.float32)

  return pl.pallas_call(
      body,
      grid=(n // r,),
      in_specs=[pl.BlockSpec((r, din), lambda i: (i, 0)),
                pl.BlockSpec((din, dout), lambda i: (0, 0))],
      out_specs=pl.BlockSpec((r, dout), lambda i: (i, 0)),
      out_shape=jax.ShapeDtypeStruct((n, dout), jnp.float32),
  )(x, w)


def _tc_dinv_g(degp, h1):
  n, d1 = h1.shape
  r = 1000

  def body(dp_ref, h_ref, dv_ref, g_ref):
    deg = dp_ref[0, :, 0:1] + dp_ref[1, :, 0:1] + 1.0
    dv = lax.rsqrt(jnp.maximum(deg, 1e-12))
    dv_ref[...] = jnp.broadcast_to(dv, dv_ref.shape)
    g_ref[...] = dv * h_ref[...]

  return pl.pallas_call(
      body,
      grid=(n // r,),
      in_specs=[pl.BlockSpec((2, r, 16), lambda i: (0, i, 0)),
                pl.BlockSpec((r, d1), lambda i: (i, 0))],
      out_specs=[pl.BlockSpec((r, 8), lambda i: (i, 0)),
                 pl.BlockSpec((r, d1), lambda i: (i, 0))],
      out_shape=[jax.ShapeDtypeStruct((n, 8), jnp.float32),
                 jax.ShapeDtypeStruct((n, d1), jnp.float32)],
  )(degp, h1)


def _tc_combine(tp, h, dinv8, b2d):
  n, d = h.shape
  r = 1000

  def body(tp_ref, h_ref, dv_ref, b_ref, o_ref, g_ref):
    dv = dv_ref[:, 0:1]
    t = tp_ref[0] + tp_ref[1]
    o = jnp.tanh(dv * t + (dv * dv) * h_ref[...] + b_ref[...])
    o_ref[...] = o
    g_ref[...] = dv * o

  return pl.pallas_call(
      body,
      grid=(n // r,),
      in_specs=[pl.BlockSpec((2, r, d), lambda i: (0, i, 0)),
                pl.BlockSpec((r, d), lambda i: (i, 0)),
                pl.BlockSpec((r, 8), lambda i: (i, 0)),
                pl.BlockSpec((1, d), lambda i: (0, 0))],
      out_specs=[pl.BlockSpec((r, d), lambda i: (i, 0)),
                 pl.BlockSpec((r, d), lambda i: (i, 0))],
      out_shape=[jax.ShapeDtypeStruct((n, d), jnp.float32),
                 jax.ShapeDtypeStruct((n, d), jnp.float32)],
  )(tp, h, dinv8, b2d)


def _tc_layer_post(tp, o_prev, dinv8, w, b2d):
  n, d = o_prev.shape
  dn = w.shape[1]
  r = 1000

  def body(tp_ref, h_ref, dv_ref, w_ref, b_ref, o_ref, g_ref):
    dv = dv_ref[:, 0:1]
    t = tp_ref[0] + tp_ref[1]
    pre = dv * t + (dv * dv) * h_ref[...]
    o = jnp.tanh(jnp.dot(pre, w_ref[...],
                         preferred_element_type=jnp.float32) + b_ref[...])
    o_ref[...] = o
    g_ref[...] = dv * o

  return pl.pallas_call(
      body,
      grid=(n // r,),
      in_specs=[pl.BlockSpec((2, r, d), lambda i: (0, i, 0)),
                pl.BlockSpec((r, d), lambda i: (i, 0)),
                pl.BlockSpec((r, 8), lambda i: (i, 0)),
                pl.BlockSpec((d, dn), lambda i: (0, 0)),
                pl.BlockSpec((1, dn), lambda i: (0, 0))],
      out_specs=[pl.BlockSpec((r, dn), lambda i: (i, 0)),
                 pl.BlockSpec((r, dn), lambda i: (i, 0))],
      out_shape=[jax.ShapeDtypeStruct((n, dn), jnp.float32),
                 jax.ShapeDtypeStruct((n, dn), jnp.float32)],
  )(tp, o_prev, dinv8, w, b2d)


def _tc_q(tp, o_prev, dinv8):
  n, d = o_prev.shape
  r = 1000

  def body(tp_ref, h_ref, dv_ref, o_ref):
    dv = dv_ref[:, 0:1]
    t = tp_ref[0] + tp_ref[1]
    o_ref[...] = dv * t + (dv * dv) * h_ref[...]

  return pl.pallas_call(
      body,
      grid=(n // r,),
      in_specs=[pl.BlockSpec((2, r, d), lambda i: (0, i, 0)),
                pl.BlockSpec((r, d), lambda i: (i, 0)),
                pl.BlockSpec((r, 8), lambda i: (i, 0))],
      out_specs=pl.BlockSpec((r, d), lambda i: (i, 0)),
      out_shape=jax.ShapeDtypeStruct((n, d), jnp.float32),
  )(tp, o_prev, dinv8)


def _tc_final(qs, cp, w4, b4_2d, wl, bl2d, g):
  mp, d = qs.shape[1], qs.shape[2]
  d4 = w4.shape[1]
  dn = wl.shape[1]

  def body(qs_ref, cp_ref, w4_ref, b4_ref, wl_ref, bl_ref, o_ref):
    qsum = qs_ref[0, :g, :] + qs_ref[1, :g, :]
    cnt = cp_ref[0, :g, 0:1] + cp_ref[1, :g, 0:1]
    maxc = jnp.maximum(cnt, 1.0)
    pooled = jnp.dot(qsum / maxc, w4_ref[...],
                     preferred_element_type=jnp.float32)
    pooled = pooled + (cnt / maxc) * b4_ref[...]
    o_ref[...] = jnp.dot(pooled, wl_ref[...],
                         preferred_element_type=jnp.float32) + bl_ref[...]

  return pl.pallas_call(
      body,
      in_specs=[pl.BlockSpec((2, mp, d), lambda: (0, 0, 0)),
                pl.BlockSpec((2, mp, 16), lambda: (0, 0, 0)),
                pl.BlockSpec((d, d4), lambda: (0, 0)),
                pl.BlockSpec((1, d4), lambda: (0, 0)),
                pl.BlockSpec((d4, dn), lambda: (0, 0)),
                pl.BlockSpec((1, dn), lambda: (0, 0))],
      out_specs=pl.BlockSpec((g, dn), lambda: (0, 0)),
      out_shape=jax.ShapeDtypeStruct((g, dn), jnp.float32),
  )(qs, cp, w4, b4_2d, wl, bl2d)


def kernel(x, edge_index, batch, W1, b1, W2, b2, W3, b3, W4, b4, Wl, bl):
  n = x.shape[0]
  e = edge_index.shape[1]
  g = 64

  ce = -(-e // (NW * EK))
  ce = ce + (ce % 2)
  e_pad = ce * NW * EK
  src_flat = jnp.concatenate(
      [edge_index[0], jnp.zeros((e_pad - e,), jnp.int32)])
  dst_flat = jnp.concatenate(
      [edge_index[1], jnp.full((e_pad - e,), n, jnp.int32)])
  src_p = src_flat.reshape(NW, ce, EK)
  dst_p = dst_flat.reshape(NW, ce, EK)
  m = -(-(n + 1) // (NS * 8)) * (NS * 8)

  cpool = -(-n // (NW * EK))
  n_pad = cpool * NW * EK
  nid_p = jnp.concatenate(
      [jnp.arange(n, dtype=jnp.int32),
       jnp.zeros((n_pad - n,), jnp.int32)]).reshape(NW, cpool, EK)
  bat_p = jnp.concatenate(
      [batch, jnp.full((n_pad - n,), g, jnp.int32)]).reshape(NW, cpool, EK)
  mp = -(-(g + 1) // (NS * 8)) * (NS * 8)

  ones16 = jnp.ones((EK, 16), jnp.float32)
  rt = m // NS
  rtp = mp // NS
  z16 = jnp.zeros((rt, 16), jnp.float32)
  zeros_d = {dd: jnp.zeros((rt, dd), jnp.float32) for dd in (16, 32, 64)}
  zp64 = jnp.zeros((rtp, 64), jnp.float32)
  zp16 = jnp.zeros((rtp, 16), jnp.float32)

  degp = _ones_scatter(ce, m)(dst_p, ones16, z16)
  h1 = _tc_matmul(x, W1)
  dinv8, g1 = _tc_dinv_g(degp, h1)

  tp1 = _edge_scatter(ce, EK, m, 16)(g1, src_p, dst_p, zeros_d[16])
  o1, g2 = _tc_combine(tp1, h1, dinv8, b1[None, :])
  tp2 = _edge_scatter(ce, EK, m, 16)(g2, src_p, dst_p, zeros_d[16])
  o2, g3 = _tc_layer_post(tp2, o1, dinv8, W2, b2[None, :])
  tp3 = _edge_scatter(ce, EK, m, 32)(g3, src_p, dst_p, zeros_d[32])
  o3, g4 = _tc_layer_post(tp3, o2, dinv8, W3, b3[None, :])
  tp4 = _edge_scatter(ce, EK, m, 64)(g4, src_p, dst_p, zeros_d[64])
  q = _tc_q(tp4, o3, dinv8)

  qs, cp = _pool_scatter(cpool, mp, 64)(
      q, nid_p, bat_p, ones16, zp64, zp16)
  return _tc_final(qs, cp, W4, b4[None, :], Wl, bl[None, :], g)


# speedup vs baseline: 36.0983x; 2.0145x over previous
import functools

import jax
import jax.numpy as jnp
from jax import lax
from jax.experimental import pallas as pl
from jax.experimental.pallas import tpu as pltpu
from jax.experimental.pallas import tpu_sc as plsc

NC = 2
NS = 16
NW = NC * NS
EK = 128

_mesh = plsc.VectorSubcoreMesh(
    core_axis_name="c", subcore_axis_name="s", num_cores=NC, num_subcores=NS)


def _edge_scatter(ce, ek, m, d, nn):
  rt = m // NS

  @functools.partial(
      pl.kernel,
      out_type=jax.ShapeDtypeStruct((NC, m, d), jnp.float32),
      mesh=_mesh,
      compiler_params=pltpu.CompilerParams(use_tc_tiling_on_sc=False),
      scratch_types=[
          pltpu.VMEM_SHARED((m, d), jnp.float32),
          pltpu.VMEM_SHARED((nn, d), jnp.float32),
          pltpu.VMEM((ce, ek), jnp.int32),
          pltpu.VMEM((ce, ek), jnp.int32),
          pltpu.VMEM((ek, d), jnp.float32),
          pltpu.VMEM((ek, d), jnp.float32),
          pltpu.SemaphoreType.DMA,
          pltpu.SemaphoreType.DMA,
      ],
  )
  def k(g_hbm, src_hbm, dst_hbm, zeros_hbm, out_hbm, acc, g_spm, src_st,
        dst_st, msg0, msg1, sem0, sem1):
    c = lax.axis_index("c")
    s = lax.axis_index("s")
    wid = c * NS + s
    n_rows = g_hbm.shape[0]
    rows8 = n_rows // 8
    pltpu.sync_copy(src_hbm.at[wid], src_st)
    pltpu.sync_copy(dst_hbm.at[wid], dst_st)
    pltpu.sync_copy(zeros_hbm, acc.at[pl.ds(s * rt, rt)])

    @pl.when(s < 8)
    def _():
      pltpu.sync_copy(g_hbm.at[pl.ds(s * rows8, rows8)],
                      g_spm.at[pl.ds(s * rows8, rows8)])

    plsc.subcore_barrier()
    pltpu.async_copy(g_spm.at[src_st.at[0]], msg0, sem0)

    msgs = (msg0, msg1)
    sems = (sem0, sem1)

    def step(gidx, carry):
      for b in (0, 1):
        j = 2 * gidx + b
        nxt = j + 1

        @pl.when(nxt < ce)
        def _():
          pltpu.async_copy(g_spm.at[src_st.at[nxt]], msgs[1 - b],
                           sems[1 - b])

        pltpu.make_async_copy(g_spm.at[pl.ds(0, ek)], msgs[b],
                              sems[b]).wait()
        pltpu.sync_copy(msgs[b], acc.at[dst_st.at[j]], add=True)
      return carry

    lax.fori_loop(0, ce // 2, step, 0)
    plsc.subcore_barrier()
    pltpu.sync_copy(acc.at[pl.ds(s * rt, rt)],
                    out_hbm.at[c, pl.ds(s * rt, rt)])

  return k


def _ones_scatter(ce, m):
  rt = m // NS

  @functools.partial(
      pl.kernel,
      out_type=jax.ShapeDtypeStruct((NC, m, 16), jnp.float32),
      mesh=_mesh,
      compiler_params=pltpu.CompilerParams(use_tc_tiling_on_sc=False),
      scratch_types=[
          pltpu.VMEM_SHARED((m, 16), jnp.float32),
          pltpu.VMEM((ce, EK), jnp.int32),
          pltpu.VMEM((EK, 16), jnp.float32),
      ],
  )
  def k(dst_hbm, ones_hbm, zeros_hbm, out_hbm, acc, dst_st, ones_st):
    c = lax.axis_index("c")
    s = lax.axis_index("s")
    wid = c * NS + s
    pltpu.sync_copy(dst_hbm.at[wid], dst_st)
    pltpu.sync_copy(zeros_hbm, acc.at[pl.ds(s * rt, rt)])
    pltpu.sync_copy(ones_hbm, ones_st)
    plsc.subcore_barrier()

    def step(j, carry):
      pltpu.sync_copy(ones_st, acc.at[dst_st.at[j]], add=True)
      return carry

    lax.fori_loop(0, ce, step, 0)
    plsc.subcore_barrier()
    pltpu.sync_copy(acc.at[pl.ds(s * rt, rt)],
                    out_hbm.at[c, pl.ds(s * rt, rt)])

  return k


def _pool_scatter(cp, mp, d, nn):
  rt = mp // NS

  @functools.partial(
      pl.kernel,
      out_type=(jax.ShapeDtypeStruct((NC, mp, d), jnp.float32),
                jax.ShapeDtypeStruct((NC, mp, 16), jnp.float32)),
      mesh=_mesh,
      compiler_params=pltpu.CompilerParams(use_tc_tiling_on_sc=False),
      scratch_types=[
          pltpu.VMEM_SHARED((mp, d), jnp.float32),
          pltpu.VMEM_SHARED((mp, 16), jnp.float32),
          pltpu.VMEM_SHARED((nn, d), jnp.float32),
          pltpu.VMEM((cp, EK), jnp.int32),
          pltpu.VMEM((cp, EK), jnp.int32),
          pltpu.VMEM((EK, d), jnp.float32),
          pltpu.VMEM((EK, 16), jnp.float32),
          pltpu.SemaphoreType.DMA,
      ],
  )
  def k(h_hbm, nid_hbm, bat_hbm, ones_hbm, zs_hbm, zc_hbm, sums_hbm,
        cnts_hbm, acc_s, acc_c, h_spm, nid_st, bat_st, msg, ones_st, sem):
    c = lax.axis_index("c")
    s = lax.axis_index("s")
    wid = c * NS + s
    n_rows = h_hbm.shape[0]
    rows8 = n_rows // 8
    pltpu.sync_copy(nid_hbm.at[wid], nid_st)
    pltpu.sync_copy(bat_hbm.at[wid], bat_st)
    pltpu.sync_copy(zs_hbm, acc_s.at[pl.ds(s * rt, rt)])
    pltpu.sync_copy(zc_hbm, acc_c.at[pl.ds(s * rt, rt)])
    pltpu.sync_copy(ones_hbm, ones_st)

    @pl.when(s < 8)
    def _():
      pltpu.sync_copy(h_hbm.at[pl.ds(s * rows8, rows8)],
                      h_spm.at[pl.ds(s * rows8, rows8)])

    plsc.subcore_barrier()

    def step(j, carry):
      pltpu.async_copy(h_spm.at[nid_st.at[j]], msg, sem).wait()
      pltpu.sync_copy(msg, acc_s.at[bat_st.at[j]], add=True)
      pltpu.sync_copy(ones_st, acc_c.at[bat_st.at[j]], add=True)
      return carry

    lax.fori_loop(0, cp, step, 0)
    plsc.subcore_barrier()
    pltpu.sync_copy(acc_s.at[pl.ds(s * rt, rt)],
                    sums_hbm.at[c, pl.ds(s * rt, rt)])
    pltpu.sync_copy(acc_c.at[pl.ds(s * rt, rt)],
                    cnts_hbm.at[c, pl.ds(s * rt, rt)])

  return k


def _tc_matmul(x, w):
  n, din = x.shape
  dout = w.shape[1]
  r = 1000

  def body(x_ref, w_ref, o_ref):
    o_ref[...] = jnp.dot(x_ref[...], w_ref[...],
                         preferred_element_type=jnp.float32)

  return pl.pallas_call(
      body,
      grid=(n // r,),
      in_specs=[pl.BlockSpec((r, din), lambda i: (i, 0)),
                pl.BlockSpec((din, dout), lambda i: (0, 0))],
      out_specs=pl.BlockSpec((r, dout), lambda i: (i, 0)),
      out_shape=jax.ShapeDtypeStruct((n, dout), jnp.float32),
  )(x, w)


def _tc_dinv_g(degp, h1):
  n, d1 = h1.shape
  r = 1000

  def body(dp_ref, h_ref, dv_ref, g_ref):
    deg = dp_ref[0, :, 0:1] + dp_ref[1, :, 0:1] + 1.0
    dv = lax.rsqrt(jnp.maximum(deg, 1e-12))
    dv_ref[...] = jnp.broadcast_to(dv, dv_ref.shape)
    g_ref[...] = dv * h_ref[...]

  return pl.pallas_call(
      body,
      grid=(n // r,),
      in_specs=[pl.BlockSpec((2, r, 16), lambda i: (0, i, 0)),
                pl.BlockSpec((r, d1), lambda i: (i, 0))],
      out_specs=[pl.BlockSpec((r, 8), lambda i: (i, 0)),
                 pl.BlockSpec((r, d1), lambda i: (i, 0))],
      out_shape=[jax.ShapeDtypeStruct((n, 8), jnp.float32),
                 jax.ShapeDtypeStruct((n, d1), jnp.float32)],
  )(degp, h1)


def _tc_combine(tp, h, dinv8, b2d):
  n, d = h.shape
  r = 1000

  def body(tp_ref, h_ref, dv_ref, b_ref, o_ref, g_ref):
    dv = dv_ref[:, 0:1]
    t = tp_ref[0] + tp_ref[1]
    o = jnp.tanh(dv * t + (dv * dv) * h_ref[...] + b_ref[...])
    o_ref[...] = o
    g_ref[...] = dv * o

  return pl.pallas_call(
      body,
      grid=(n // r,),
      in_specs=[pl.BlockSpec((2, r, d), lambda i: (0, i, 0)),
                pl.BlockSpec((r, d), lambda i: (i, 0)),
                pl.BlockSpec((r, 8), lambda i: (i, 0)),
                pl.BlockSpec((1, d), lambda i: (0, 0))],
      out_specs=[pl.BlockSpec((r, d), lambda i: (i, 0)),
                 pl.BlockSpec((r, d), lambda i: (i, 0))],
      out_shape=[jax.ShapeDtypeStruct((n, d), jnp.float32),
                 jax.ShapeDtypeStruct((n, d), jnp.float32)],
  )(tp, h, dinv8, b2d)


def _tc_layer_post(tp, o_prev, dinv8, w, b2d):
  n, d = o_prev.shape
  dn = w.shape[1]
  r = 1000

  def body(tp_ref, h_ref, dv_ref, w_ref, b_ref, o_ref, g_ref):
    dv = dv_ref[:, 0:1]
    t = tp_ref[0] + tp_ref[1]
    pre = dv * t + (dv * dv) * h_ref[...]
    o = jnp.tanh(jnp.dot(pre, w_ref[...],
                         preferred_element_type=jnp.float32) + b_ref[...])
    o_ref[...] = o
    g_ref[...] = dv * o

  return pl.pallas_call(
      body,
      grid=(n // r,),
      in_specs=[pl.BlockSpec((2, r, d), lambda i: (0, i, 0)),
                pl.BlockSpec((r, d), lambda i: (i, 0)),
                pl.BlockSpec((r, 8), lambda i: (i, 0)),
                pl.BlockSpec((d, dn), lambda i: (0, 0)),
                pl.BlockSpec((1, dn), lambda i: (0, 0))],
      out_specs=[pl.BlockSpec((r, dn), lambda i: (i, 0)),
                 pl.BlockSpec((r, dn), lambda i: (i, 0))],
      out_shape=[jax.ShapeDtypeStruct((n, dn), jnp.float32),
                 jax.ShapeDtypeStruct((n, dn), jnp.float32)],
  )(tp, o_prev, dinv8, w, b2d)


def _tc_q(tp, o_prev, dinv8):
  n, d = o_prev.shape
  r = 1000

  def body(tp_ref, h_ref, dv_ref, o_ref):
    dv = dv_ref[:, 0:1]
    t = tp_ref[0] + tp_ref[1]
    o_ref[...] = dv * t + (dv * dv) * h_ref[...]

  return pl.pallas_call(
      body,
      grid=(n // r,),
      in_specs=[pl.BlockSpec((2, r, d), lambda i: (0, i, 0)),
                pl.BlockSpec((r, d), lambda i: (i, 0)),
                pl.BlockSpec((r, 8), lambda i: (i, 0))],
      out_specs=pl.BlockSpec((r, d), lambda i: (i, 0)),
      out_shape=jax.ShapeDtypeStruct((n, d), jnp.float32),
  )(tp, o_prev, dinv8)


def _tc_final(qs, cp, w4, b4_2d, wl, bl2d, g):
  mp, d = qs.shape[1], qs.shape[2]
  d4 = w4.shape[1]
  dn = wl.shape[1]

  def body(qs_ref, cp_ref, w4_ref, b4_ref, wl_ref, bl_ref, o_ref):
    qsum = qs_ref[0, :g, :] + qs_ref[1, :g, :]
    cnt = cp_ref[0, :g, 0:1] + cp_ref[1, :g, 0:1]
    maxc = jnp.maximum(cnt, 1.0)
    pooled = jnp.dot(qsum / maxc, w4_ref[...],
                     preferred_element_type=jnp.float32)
    pooled = pooled + (cnt / maxc) * b4_ref[...]
    o_ref[...] = jnp.dot(pooled, wl_ref[...],
                         preferred_element_type=jnp.float32) + bl_ref[...]

  return pl.pallas_call(
      body,
      in_specs=[pl.BlockSpec((2, mp, d), lambda: (0, 0, 0)),
                pl.BlockSpec((2, mp, 16), lambda: (0, 0, 0)),
                pl.BlockSpec((d, d4), lambda: (0, 0)),
                pl.BlockSpec((1, d4), lambda: (0, 0)),
                pl.BlockSpec((d4, dn), lambda: (0, 0)),
                pl.BlockSpec((1, dn), lambda: (0, 0))],
      out_specs=pl.BlockSpec((g, dn), lambda: (0, 0)),
      out_shape=jax.ShapeDtypeStruct((g, dn), jnp.float32),
  )(qs, cp, w4, b4_2d, wl, bl2d)


def kernel(x, edge_index, batch, W1, b1, W2, b2, W3, b3, W4, b4, Wl, bl):
  n = x.shape[0]
  e = edge_index.shape[1]
  g = 64

  ce = -(-e // (NW * EK))
  ce = ce + (ce % 2)
  e_pad = ce * NW * EK
  src_flat = jnp.concatenate(
      [edge_index[0], jnp.zeros((e_pad - e,), jnp.int32)])
  dst_flat = jnp.concatenate(
      [edge_index[1], jnp.full((e_pad - e,), n, jnp.int32)])
  src_p = src_flat.reshape(NW, ce, EK)
  dst_p = dst_flat.reshape(NW, ce, EK)
  m = -(-(n + 1) // (NS * 8)) * (NS * 8)

  cpool = -(-n // (NW * EK))
  n_pad = cpool * NW * EK
  nid_p = jnp.concatenate(
      [jnp.arange(n, dtype=jnp.int32),
       jnp.zeros((n_pad - n,), jnp.int32)]).reshape(NW, cpool, EK)
  bat_p = jnp.concatenate(
      [batch, jnp.full((n_pad - n,), g, jnp.int32)]).reshape(NW, cpool, EK)
  mp = -(-(g + 1) // (NS * 8)) * (NS * 8)

  ones16 = jnp.ones((EK, 16), jnp.float32)
  rt = m // NS
  rtp = mp // NS
  z16 = jnp.zeros((rt, 16), jnp.float32)
  zeros_d = {dd: jnp.zeros((rt, dd), jnp.float32) for dd in (16, 32, 64)}
  zp64 = jnp.zeros((rtp, 64), jnp.float32)
  zp16 = jnp.zeros((rtp, 16), jnp.float32)

  degp = _ones_scatter(ce, m)(dst_p, ones16, z16)
  h1 = _tc_matmul(x, W1)
  dinv8, g1 = _tc_dinv_g(degp, h1)

  tp1 = _edge_scatter(ce, EK, m, 16, n)(g1, src_p, dst_p, zeros_d[16])
  o1, g2 = _tc_combine(tp1, h1, dinv8, b1[None, :])
  tp2 = _edge_scatter(ce, EK, m, 16, n)(g2, src_p, dst_p, zeros_d[16])
  o2, g3 = _tc_layer_post(tp2, o1, dinv8, W2, b2[None, :])
  tp3 = _edge_scatter(ce, EK, m, 32, n)(g3, src_p, dst_p, zeros_d[32])
  o3, g4 = _tc_layer_post(tp3, o2, dinv8, W3, b3[None, :])
  tp4 = _edge_scatter(ce, EK, m, 64, n)(g4, src_p, dst_p, zeros_d[64])
  q = _tc_q(tp4, o3, dinv8)

  qs, cp = _pool_scatter(cpool, mp, 64, n)(
      q, nid_p, bat_p, ones16, zp64, zp16)
  return _tc_final(qs, cp, W4, b4[None, :], Wl, bl[None, :], g)


# speedup vs baseline: 39.5685x; 1.0961x over previous
import functools

import jax
import jax.numpy as jnp
from jax import lax
from jax.experimental import pallas as pl
from jax.experimental.pallas import tpu as pltpu
from jax.experimental.pallas import tpu_sc as plsc

NC = 2
NS = 16
NW = NC * NS
EK = 128

_mesh = plsc.VectorSubcoreMesh(
    core_axis_name="c", subcore_axis_name="s", num_cores=NC, num_subcores=NS)
_params = pltpu.CompilerParams(use_tc_tiling_on_sc=False)


def _hist_kernel(ce, m, cp, mp):
  rt = m // NS
  rtp = mp // NS

  @functools.partial(
      pl.kernel,
      out_type=(jax.ShapeDtypeStruct((NC, m, 16), jnp.float32),
                jax.ShapeDtypeStruct((NC, mp, 16), jnp.float32)),
      mesh=_mesh,
      compiler_params=_params,
      scratch_types=[
          pltpu.VMEM_SHARED((m, 16), jnp.float32),
          pltpu.VMEM_SHARED((mp, 16), jnp.float32),
          pltpu.VMEM((ce, EK), jnp.int32),
          pltpu.VMEM((cp, EK), jnp.int32),
          pltpu.VMEM((EK, 16), jnp.float32),
      ],
  )
  def k(dst_hbm, bat_hbm, ones_hbm, zd_hbm, zc_hbm, deg_hbm, cnt_hbm,
        acc_d, acc_c, dst_st, bat_st, ones_st):
    c = lax.axis_index("c")
    s = lax.axis_index("s")
    wid = c * NS + s
    pltpu.sync_copy(dst_hbm.at[wid], dst_st)
    pltpu.sync_copy(bat_hbm.at[wid], bat_st)
    pltpu.sync_copy(zd_hbm, acc_d.at[pl.ds(s * rt, rt)])
    pltpu.sync_copy(zc_hbm, acc_c.at[pl.ds(s * rtp, rtp)])
    pltpu.sync_copy(ones_hbm, ones_st)
    plsc.subcore_barrier()

    def step(j, carry):
      pltpu.sync_copy(ones_st, acc_d.at[dst_st.at[j]], add=True)
      return carry

    lax.fori_loop(0, ce, step, 0)

    def stepb(j, carry):
      pltpu.sync_copy(ones_st, acc_c.at[bat_st.at[j]], add=True)
      return carry

    lax.fori_loop(0, cp, stepb, 0)
    plsc.subcore_barrier()
    pltpu.sync_copy(acc_d.at[pl.ds(s * rt, rt)],
                    deg_hbm.at[c, pl.ds(s * rt, rt)])
    pltpu.sync_copy(acc_c.at[pl.ds(s * rtp, rtp)],
                    cnt_hbm.at[c, pl.ds(s * rtp, rtp)])

  return k


def _edge_body(ce, m, d, nn, g_hbm, src_hbm, dst_hbm, zeros_hbm, acc,
               g_spm, src_st, dst_st, msg0, msg1, sem0, sem1, c, s):
  rt = m // NS
  wid = c * NS + s
  rows8 = nn // 8
  tail = nn - (NS - 1) * rt
  pltpu.sync_copy(src_hbm.at[wid], src_st)
  pltpu.sync_copy(dst_hbm.at[wid], dst_st)

  @pl.when(jnp.logical_and(c == 0, s < NS - 1))
  def _():
    pltpu.sync_copy(g_hbm.at[pl.ds(s * rt, rt)], acc.at[pl.ds(s * rt, rt)])

  @pl.when(jnp.logical_and(c == 0, s == NS - 1))
  def _():
    pltpu.sync_copy(zeros_hbm, acc.at[pl.ds((NS - 1) * rt, rt)])
    pltpu.sync_copy(g_hbm.at[pl.ds((NS - 1) * rt, tail)],
                    acc.at[pl.ds((NS - 1) * rt, tail)])

  @pl.when(c == 1)
  def _():
    pltpu.sync_copy(zeros_hbm, acc.at[pl.ds(s * rt, rt)])

  @pl.when(s < 8)
  def _():
    pltpu.sync_copy(g_hbm.at[pl.ds(s * rows8, rows8)],
                    g_spm.at[pl.ds(s * rows8, rows8)])

  plsc.subcore_barrier()
  pltpu.async_copy(g_spm.at[src_st.at[0]], msg0, sem0)

  msgs = (msg0, msg1)
  sems = (sem0, sem1)

  def step(gidx, carry):
    for b in (0, 1):
      j = 2 * gidx + b
      nxt = j + 1

      @pl.when(nxt < ce)
      def _():
        pltpu.async_copy(g_spm.at[src_st.at[nxt]], msgs[1 - b],
                         sems[1 - b])

      pltpu.make_async_copy(g_spm.at[pl.ds(0, EK)], msgs[b],
                            sems[b]).wait()
      pltpu.sync_copy(msgs[b], acc.at[dst_st.at[j]], add=True)
    return carry

  lax.fori_loop(0, ce // 2, step, 0)
  plsc.subcore_barrier()


def _edge_scatter(ce, m, d, nn):
  rt = m // NS

  @functools.partial(
      pl.kernel,
      out_type=jax.ShapeDtypeStruct((NC, m, d), jnp.float32),
      mesh=_mesh,
      compiler_params=_params,
      scratch_types=[
          pltpu.VMEM_SHARED((m, d), jnp.float32),
          pltpu.VMEM_SHARED((nn, d), jnp.float32),
          pltpu.VMEM((ce, EK), jnp.int32),
          pltpu.VMEM((ce, EK), jnp.int32),
          pltpu.VMEM((EK, d), jnp.float32),
          pltpu.VMEM((EK, d), jnp.float32),
          pltpu.SemaphoreType.DMA,
          pltpu.SemaphoreType.DMA,
      ],
  )
  def k(g_hbm, src_hbm, dst_hbm, zeros_hbm, out_hbm, acc, g_spm, src_st,
        dst_st, msg0, msg1, sem0, sem1):
    c = lax.axis_index("c")
    s = lax.axis_index("s")
    _edge_body(ce, m, d, nn, g_hbm, src_hbm, dst_hbm, zeros_hbm, acc,
               g_spm, src_st, dst_st, msg0, msg1, sem0, sem1, c, s)
    pltpu.sync_copy(acc.at[pl.ds(s * rt, rt)],
                    out_hbm.at[c, pl.ds(s * rt, rt)])

  return k


def _edge_pool_scatter(ce, m, d, nn, mp, zk):
  rt = m // NS
  rtp = mp // NS
  nz = rt // zk

  @functools.partial(
      pl.kernel,
      out_type=jax.ShapeDtypeStruct((NC, mp, d), jnp.float32),
      mesh=_mesh,
      compiler_params=_params,
      scratch_types=[
          pltpu.VMEM_SHARED((m, d), jnp.float32),
          pltpu.VMEM_SHARED((nn, d), jnp.float32),
          pltpu.VMEM_SHARED((mp, d), jnp.float32),
          pltpu.VMEM((ce, EK), jnp.int32),
          pltpu.VMEM((ce, EK), jnp.int32),
          pltpu.VMEM((EK, d), jnp.float32),
          pltpu.VMEM((EK, d), jnp.float32),
          pltpu.VMEM((zk, d), jnp.float32),
          pltpu.VMEM((zk, 16), jnp.float32),
          pltpu.VMEM((nz, zk), jnp.int32),
          pltpu.SemaphoreType.DMA,
          pltpu.SemaphoreType.DMA,
      ],
  )
  def k(g_hbm, src_hbm, dst_hbm, zeros_hbm, dinv_hbm, batz_hbm, zp_hbm,
        pool_hbm, acc, g_spm, acc_p, src_st, dst_st, msg0, msg1, z_st,
        dv_st, bat_st, sem0, sem1):
    c = lax.axis_index("c")
    s = lax.axis_index("s")
    pltpu.sync_copy(batz_hbm.at[s], bat_st)
    pltpu.sync_copy(zp_hbm, acc_p.at[pl.ds(s * rtp, rtp)])
    _edge_body(ce, m, d, nn, g_hbm, src_hbm, dst_hbm, zeros_hbm, acc,
               g_spm, src_st, dst_st, msg0, msg1, sem0, sem1, c, s)

    def pool(j, carry):
      pltpu.sync_copy(acc.at[pl.ds(s * rt + j * zk, zk)], z_st)
      pltpu.sync_copy(dinv_hbm.at[pl.ds(s * rt + j * zk, zk)], dv_st)

      def scale(r, carry2):
        dv = dv_st[r, :]
        for cg in range(d // 16):
          z_st[r, pl.ds(cg * 16, 16)] = z_st[r, pl.ds(cg * 16, 16)] * dv
        return carry2

      lax.fori_loop(0, zk, scale, 0)
      pltpu.sync_copy(z_st, acc_p.at[bat_st.at[j]], add=True)
      return carry

    lax.fori_loop(0, nz, pool, 0)
    plsc.subcore_barrier()
    pltpu.sync_copy(acc_p.at[pl.ds(s * rtp, rtp)],
                    pool_hbm.at[c, pl.ds(s * rtp, rtp)])

  return k


def _tc_front(degp, x, w1):
  n, din = x.shape
  d1 = w1.shape[1]
  r = 1000

  def body(dp_ref, x_ref, w_ref, dv8_ref, dv16_ref, g_ref):
    deg = dp_ref[0, :, 0:1] + dp_ref[1, :, 0:1] + 1.0
    dv = lax.rsqrt(jnp.maximum(deg, 1e-12))
    dv8_ref[...] = jnp.broadcast_to(dv, dv8_ref.shape)
    dv16_ref[...] = jnp.broadcast_to(dv, dv16_ref.shape)
    h1 = jnp.dot(x_ref[...], w_ref[...], preferred_element_type=jnp.float32)
    g_ref[...] = dv * h1

  return pl.pallas_call(
      body,
      grid=(n // r,),
      in_specs=[pl.BlockSpec((2, r, 16), lambda i: (0, i, 0)),
                pl.BlockSpec((r, din), lambda i: (i, 0)),
                pl.BlockSpec((din, d1), lambda i: (0, 0))],
      out_specs=[pl.BlockSpec((r, 8), lambda i: (i, 0)),
                 pl.BlockSpec((r, 16), lambda i: (i, 0)),
                 pl.BlockSpec((r, d1), lambda i: (i, 0))],
      out_shape=[jax.ShapeDtypeStruct((n, 8), jnp.float32),
                 jax.ShapeDtypeStruct((n, 16), jnp.float32),
                 jax.ShapeDtypeStruct((n, d1), jnp.float32)],
  )(degp, x, w1)


def _tc_layer1(tp, dinv8, b1_2d):
  n = dinv8.shape[0]
  d = tp.shape[2]
  r = 1000

  def body(tp_ref, dv_ref, b_ref, g_ref):
    dv = dv_ref[:, 0:1]
    o = jnp.tanh(dv * (tp_ref[0] + tp_ref[1]) + b_ref[...])
    g_ref[...] = dv * o

  return pl.pallas_call(
      body,
      grid=(n // r,),
      in_specs=[pl.BlockSpec((2, r, d), lambda i: (0, i, 0)),
                pl.BlockSpec((r, 8), lambda i: (i, 0)),
                pl.BlockSpec((1, d), lambda i: (0, 0))],
      out_specs=pl.BlockSpec((r, d), lambda i: (i, 0)),
      out_shape=jax.ShapeDtypeStruct((n, d), jnp.float32),
  )(tp, dinv8, b1_2d)


def _tc_layer(tp, dinv8, w, b2d):
  n = dinv8.shape[0]
  d = tp.shape[2]
  dn = w.shape[1]
  r = 1000

  def body(tp_ref, dv_ref, w_ref, b_ref, g_ref):
    dv = dv_ref[:, 0:1]
    t = dv * (tp_ref[0] + tp_ref[1])
    o = jnp.tanh(jnp.dot(t, w_ref[...],
                         preferred_element_type=jnp.float32) + b_ref[...])
    g_ref[...] = dv * o

  return pl.pallas_call(
      body,
      grid=(n // r,),
      in_specs=[pl.BlockSpec((2, r, d), lambda i: (0, i, 0)),
                pl.BlockSpec((r, 8), lambda i: (i, 0)),
                pl.BlockSpec((d, dn), lambda i: (0, 0)),
                pl.BlockSpec((1, dn), lambda i: (0, 0))],
      out_specs=pl.BlockSpec((r, dn), lambda i: (i, 0)),
      out_shape=jax.ShapeDtypeStruct((n, dn), jnp.float32),
  )(tp, dinv8, w, b2d)


def _tc_final(qs, cp, w4, b4_2d, wl, bl2d, g):
  mp, d = qs.shape[1], qs.shape[2]
  mpc = cp.shape[1]
  d4 = w4.shape[1]
  dn = wl.shape[1]

  def body(qs_ref, cp_ref, w4_ref, b4_ref, wl_ref, bl_ref, o_ref):
    qsum = qs_ref[0, :g, :] + qs_ref[1, :g, :]
    cnt = cp_ref[0, :g, 0:1] + cp_ref[1, :g, 0:1]
    maxc = jnp.maximum(cnt, 1.0)
    pooled = jnp.dot(qsum / maxc, w4_ref[...],
                     preferred_element_type=jnp.float32)
    pooled = pooled + (cnt / maxc) * b4_ref[...]
    o_ref[...] = jnp.dot(pooled, wl_ref[...],
                         preferred_element_type=jnp.float32) + bl_ref[...]

  return pl.pallas_call(
      body,
      in_specs=[pl.BlockSpec((2, mp, d), lambda: (0, 0, 0)),
                pl.BlockSpec((2, mpc, 16), lambda: (0, 0, 0)),
                pl.BlockSpec((d, d4), lambda: (0, 0)),
                pl.BlockSpec((1, d4), lambda: (0, 0)),
                pl.BlockSpec((d4, dn), lambda: (0, 0)),
                pl.BlockSpec((1, dn), lambda: (0, 0))],
      out_specs=pl.BlockSpec((g, dn), lambda: (0, 0)),
      out_shape=jax.ShapeDtypeStruct((g, dn), jnp.float32),
  )(qs, cp, w4, b4_2d, wl, bl2d)


def kernel(x, edge_index, batch, W1, b1, W2, b2, W3, b3, W4, b4, Wl, bl):
  n = x.shape[0]
  e = edge_index.shape[1]
  g = 64

  ce = -(-e // (NW * EK))
  ce = ce + (ce % 2)
  e_pad = ce * NW * EK
  src_p = jnp.concatenate(
      [edge_index[0], jnp.zeros((e_pad - e,), jnp.int32)]).reshape(
          NW, ce, EK)
  dst_p = jnp.concatenate(
      [edge_index[1], jnp.full((e_pad - e,), n, jnp.int32)]).reshape(
          NW, ce, EK)
  m = -(-(n + 1) // (NS * 64)) * (NS * 64)
  rt = m // NS

  cpool = -(-n // (NW * EK))
  n_pad = cpool * NW * EK
  bat_p = jnp.concatenate(
      [batch, jnp.full((n_pad - n,), g, jnp.int32)]).reshape(NW, cpool, EK)
  mp = -(-(g + 1) // (NS * 8)) * (NS * 8)
  rtp = mp // NS

  zk = rt // 8
  bat_z = jnp.concatenate(
      [batch, jnp.full((m - n,), g, jnp.int32)]).reshape(NS, rt // zk, zk)

  ones16 = jnp.ones((EK, 16), jnp.float32)
  z16 = jnp.zeros((rt, 16), jnp.float32)
  zc16 = jnp.zeros((rtp, 16), jnp.float32)
  zeros_d = {dd: jnp.zeros((rt, dd), jnp.float32) for dd in (16, 32, 64)}
  zp64 = jnp.zeros((rtp, 64), jnp.float32)

  degp, cntp = _hist_kernel(ce, m, cpool, mp)(
      dst_p, bat_p, ones16, z16, zc16)
  dinv8, dinv16, g1 = _tc_front(degp, x, W1)
  dinv16m = jnp.concatenate(
      [dinv16, jnp.zeros((m - n, 16), jnp.float32)])

  tp1 = _edge_scatter(ce, m, 16, n)(g1, src_p, dst_p, zeros_d[16])
  g2 = _tc_layer1(tp1, dinv8, b1[None, :])
  tp2 = _edge_scatter(ce, m, 16, n)(g2, src_p, dst_p, zeros_d[16])
  g3 = _tc_layer(tp2, dinv8, W2, b2[None, :])
  tp3 = _edge_scatter(ce, m, 32, n)(g3, src_p, dst_p, zeros_d[32])
  g4 = _tc_layer(tp3, dinv8, W3, b3[None, :])

  qs = _edge_pool_scatter(ce, m, 64, n, mp, zk)(
      g4, src_p, dst_p, zeros_d[64], dinv16m, bat_z, zp64)
  return _tc_final(qs, cntp, W4, b4[None, :], Wl, bl[None, :], g)


# speedup vs baseline: 39.5766x; 1.0002x over previous
import functools

import jax
import jax.numpy as jnp
from jax import lax
from jax.experimental import pallas as pl
from jax.experimental.pallas import tpu as pltpu
from jax.experimental.pallas import tpu_sc as plsc

NC = 2
NS = 16
NW = NC * NS
EK = 128

_mesh = plsc.VectorSubcoreMesh(
    core_axis_name="c", subcore_axis_name="s", num_cores=NC, num_subcores=NS)
_params = pltpu.CompilerParams(use_tc_tiling_on_sc=False)


def _hist_kernel(ce, m, cp, mp):
  rt = m // NS
  rtp = mp // NS

  @functools.partial(
      pl.kernel,
      out_type=(jax.ShapeDtypeStruct((NC, m, 16), jnp.float32),
                jax.ShapeDtypeStruct((NC, mp, 16), jnp.float32)),
      mesh=_mesh,
      compiler_params=_params,
      scratch_types=[
          pltpu.VMEM_SHARED((m, 16), jnp.float32),
          pltpu.VMEM_SHARED((mp, 16), jnp.float32),
          pltpu.VMEM((ce, EK), jnp.int32),
          pltpu.VMEM((cp, EK), jnp.int32),
          pltpu.VMEM((EK, 16), jnp.float32),
      ],
  )
  def k(dst_hbm, bat_hbm, ones_hbm, zd_hbm, zc_hbm, deg_hbm, cnt_hbm,
        acc_d, acc_c, dst_st, bat_st, ones_st):
    c = lax.axis_index("c")
    s = lax.axis_index("s")
    wid = c * NS + s
    pltpu.sync_copy(dst_hbm.at[wid], dst_st)
    pltpu.sync_copy(bat_hbm.at[wid], bat_st)
    pltpu.sync_copy(zd_hbm, acc_d.at[pl.ds(s * rt, rt)])
    pltpu.sync_copy(zc_hbm, acc_c.at[pl.ds(s * rtp, rtp)])
    pltpu.sync_copy(ones_hbm, ones_st)
    plsc.subcore_barrier()

    def step(j, carry):
      pltpu.sync_copy(ones_st, acc_d.at[dst_st.at[j]], add=True)
      return carry

    lax.fori_loop(0, ce, step, 0)

    def stepb(j, carry):
      pltpu.sync_copy(ones_st, acc_c.at[bat_st.at[j]], add=True)
      return carry

    lax.fori_loop(0, cp, stepb, 0)
    plsc.subcore_barrier()
    pltpu.sync_copy(acc_d.at[pl.ds(s * rt, rt)],
                    deg_hbm.at[c, pl.ds(s * rt, rt)])
    pltpu.sync_copy(acc_c.at[pl.ds(s * rtp, rtp)],
                    cnt_hbm.at[c, pl.ds(s * rtp, rtp)])

  return k


def _edge_body(ce, m, d, nn, g_hbm, src_hbm, dst_hbm, zeros_hbm, acc,
               g_spm, src_st, dst_st, msg0, msg1, sem0, sem1, ssem0,
               ssem1, c, s):
  rt = m // NS
  wid = c * NS + s
  rows8 = nn // 8
  tail = nn - (NS - 1) * rt
  pltpu.sync_copy(src_hbm.at[wid], src_st)
  pltpu.sync_copy(dst_hbm.at[wid], dst_st)

  @pl.when(jnp.logical_and(c == 0, s < NS - 1))
  def _():
    pltpu.sync_copy(g_hbm.at[pl.ds(s * rt, rt)], acc.at[pl.ds(s * rt, rt)])

  @pl.when(jnp.logical_and(c == 0, s == NS - 1))
  def _():
    pltpu.sync_copy(zeros_hbm, acc.at[pl.ds((NS - 1) * rt, rt)])
    pltpu.sync_copy(g_hbm.at[pl.ds((NS - 1) * rt, tail)],
                    acc.at[pl.ds((NS - 1) * rt, tail)])

  @pl.when(c == 1)
  def _():
    pltpu.sync_copy(zeros_hbm, acc.at[pl.ds(s * rt, rt)])

  @pl.when(s < 8)
  def _():
    pltpu.sync_copy(g_hbm.at[pl.ds(s * rows8, rows8)],
                    g_spm.at[pl.ds(s * rows8, rows8)])

  plsc.subcore_barrier()
  pltpu.async_copy(g_spm.at[src_st.at[0]], msg0, sem0)

  msgs = (msg0, msg1)
  sems = (sem0, sem1)
  ssems = (ssem0, ssem1)

  def step(gidx, carry):
    for b in (0, 1):
      j = 2 * gidx + b
      nxt = j + 1

      @pl.when(j >= 1)
      def _():
        pltpu.make_async_copy(msgs[1 - b], acc.at[pl.ds(0, EK)],
                              ssems[1 - b]).wait()

      @pl.when(nxt < ce)
      def _():
        pltpu.async_copy(g_spm.at[src_st.at[nxt]], msgs[1 - b],
                         sems[1 - b])

      pltpu.make_async_copy(g_spm.at[pl.ds(0, EK)], msgs[b],
                            sems[b]).wait()
      pltpu.async_copy(msgs[b], acc.at[dst_st.at[j]], ssems[b], add=True)
    return carry

  lax.fori_loop(0, ce // 2, step, 0)
  pltpu.make_async_copy(msgs[1], acc.at[pl.ds(0, EK)], ssems[1]).wait()
  plsc.subcore_barrier()


def _edge_scatter(ce, m, d, nn):
  rt = m // NS

  @functools.partial(
      pl.kernel,
      out_type=jax.ShapeDtypeStruct((NC, m, d), jnp.float32),
      mesh=_mesh,
      compiler_params=_params,
      scratch_types=[
          pltpu.VMEM_SHARED((m, d), jnp.float32),
          pltpu.VMEM_SHARED((nn, d), jnp.float32),
          pltpu.VMEM((ce, EK), jnp.int32),
          pltpu.VMEM((ce, EK), jnp.int32),
          pltpu.VMEM((EK, d), jnp.float32),
          pltpu.VMEM((EK, d), jnp.float32),
          pltpu.SemaphoreType.DMA,
          pltpu.SemaphoreType.DMA,
          pltpu.SemaphoreType.DMA,
          pltpu.SemaphoreType.DMA,
      ],
  )
  def k(g_hbm, src_hbm, dst_hbm, zeros_hbm, out_hbm, acc, g_spm, src_st,
        dst_st, msg0, msg1, sem0, sem1, ssem0, ssem1):
    c = lax.axis_index("c")
    s = lax.axis_index("s")
    _edge_body(ce, m, d, nn, g_hbm, src_hbm, dst_hbm, zeros_hbm, acc,
               g_spm, src_st, dst_st, msg0, msg1, sem0, sem1, ssem0,
               ssem1, c, s)
    pltpu.sync_copy(acc.at[pl.ds(s * rt, rt)],
                    out_hbm.at[c, pl.ds(s * rt, rt)])

  return k


def _edge_pool_scatter(ce, m, d, nn, mp, zk):
  rt = m // NS
  rtp = mp // NS
  nz = rt // zk

  @functools.partial(
      pl.kernel,
      out_type=jax.ShapeDtypeStruct((NC, mp, d), jnp.float32),
      mesh=_mesh,
      compiler_params=_params,
      scratch_types=[
          pltpu.VMEM_SHARED((m, d), jnp.float32),
          pltpu.VMEM_SHARED((nn, d), jnp.float32),
          pltpu.VMEM_SHARED((mp, d), jnp.float32),
          pltpu.VMEM((ce, EK), jnp.int32),
          pltpu.VMEM((ce, EK), jnp.int32),
          pltpu.VMEM((EK, d), jnp.float32),
          pltpu.VMEM((EK, d), jnp.float32),
          pltpu.VMEM((zk, d), jnp.float32),
          pltpu.VMEM((zk, 16), jnp.float32),
          pltpu.VMEM((nz, zk), jnp.int32),
          pltpu.SemaphoreType.DMA,
          pltpu.SemaphoreType.DMA,
          pltpu.SemaphoreType.DMA,
          pltpu.SemaphoreType.DMA,
      ],
  )
  def k(g_hbm, src_hbm, dst_hbm, zeros_hbm, dinv_hbm, batz_hbm, zp_hbm,
        pool_hbm, acc, g_spm, acc_p, src_st, dst_st, msg0, msg1, z_st,
        dv_st, bat_st, sem0, sem1, ssem0, ssem1):
    c = lax.axis_index("c")
    s = lax.axis_index("s")
    pltpu.sync_copy(batz_hbm.at[s], bat_st)
    pltpu.sync_copy(zp_hbm, acc_p.at[pl.ds(s * rtp, rtp)])
    _edge_body(ce, m, d, nn, g_hbm, src_hbm, dst_hbm, zeros_hbm, acc,
               g_spm, src_st, dst_st, msg0, msg1, sem0, sem1, ssem0,
               ssem1, c, s)

    def pool(j, carry):
      pltpu.sync_copy(acc.at[pl.ds(s * rt + j * zk, zk)], z_st)
      pltpu.sync_copy(dinv_hbm.at[pl.ds(s * rt + j * zk, zk)], dv_st)

      def scale(r, carry2):
        dv = dv_st[r, :]
        for cg in range(d // 16):
          z_st[r, pl.ds(cg * 16, 16)] = z_st[r, pl.ds(cg * 16, 16)] * dv
        return carry2

      lax.fori_loop(0, zk, scale, 0)
      pltpu.sync_copy(z_st, acc_p.at[bat_st.at[j]], add=True)
      return carry

    lax.fori_loop(0, nz, pool, 0)
    plsc.subcore_barrier()
    pltpu.sync_copy(acc_p.at[pl.ds(s * rtp, rtp)],
                    pool_hbm.at[c, pl.ds(s * rtp, rtp)])

  return k


def _tc_matmul(x, w):
  n, din = x.shape
  dout = w.shape[1]
  r = 1000

  def body(x_ref, w_ref, o_ref):
    o_ref[...] = jnp.dot(x_ref[...], w_ref[...],
                         preferred_element_type=jnp.float32)

  return pl.pallas_call(
      body,
      grid=(n // r,),
      in_specs=[pl.BlockSpec((r, din), lambda i: (i, 0)),
                pl.BlockSpec((din, dout), lambda i: (0, 0))],
      out_specs=pl.BlockSpec((r, dout), lambda i: (i, 0)),
      out_shape=jax.ShapeDtypeStruct((n, dout), jnp.float32),
  )(x, w)


def _tc_dinv_g(degp, h1):
  n, d1 = h1.shape
  r = 1000

  def body(dp_ref, h_ref, dv8_ref, dv16_ref, g_ref):
    deg = dp_ref[0, :, 0:1] + dp_ref[1, :, 0:1] + 1.0
    dv = lax.rsqrt(jnp.maximum(deg, 1e-12))
    dv8_ref[...] = jnp.broadcast_to(dv, dv8_ref.shape)
    dv16_ref[...] = jnp.broadcast_to(dv, dv16_ref.shape)
    g_ref[...] = dv * h_ref[...]

  return pl.pallas_call(
      body,
      grid=(n // r,),
      in_specs=[pl.BlockSpec((2, r, 16), lambda i: (0, i, 0)),
                pl.BlockSpec((r, d1), lambda i: (i, 0))],
      out_specs=[pl.BlockSpec((r, 8), lambda i: (i, 0)),
                 pl.BlockSpec((r, 16), lambda i: (i, 0)),
                 pl.BlockSpec((r, d1), lambda i: (i, 0))],
      out_shape=[jax.ShapeDtypeStruct((n, 8), jnp.float32),
                 jax.ShapeDtypeStruct((n, 16), jnp.float32),
                 jax.ShapeDtypeStruct((n, d1), jnp.float32)],
  )(degp, h1)


def _tc_layer1(tp, dinv8, b1_2d):
  n = dinv8.shape[0]
  d = tp.shape[2]
  r = 1000

  def body(tp_ref, dv_ref, b_ref, g_ref):
    dv = dv_ref[:, 0:1]
    o = jnp.tanh(dv * (tp_ref[0] + tp_ref[1]) + b_ref[...])
    g_ref[...] = dv * o

  return pl.pallas_call(
      body,
      grid=(n // r,),
      in_specs=[pl.BlockSpec((2, r, d), lambda i: (0, i, 0)),
                pl.BlockSpec((r, 8), lambda i: (i, 0)),
                pl.BlockSpec((1, d), lambda i: (0, 0))],
      out_specs=pl.BlockSpec((r, d), lambda i: (i, 0)),
      out_shape=jax.ShapeDtypeStruct((n, d), jnp.float32),
  )(tp, dinv8, b1_2d)


def _tc_layer(tp, dinv8, w, b2d):
  n = dinv8.shape[0]
  d = tp.shape[2]
  dn = w.shape[1]
  r = 1000

  def body(tp_ref, dv_ref, w_ref, b_ref, g_ref):
    dv = dv_ref[:, 0:1]
    t = dv * (tp_ref[0] + tp_ref[1])
    o = jnp.tanh(jnp.dot(t, w_ref[...],
                         preferred_element_type=jnp.float32) + b_ref[...])
    g_ref[...] = dv * o

  return pl.pallas_call(
      body,
      grid=(n // r,),
      in_specs=[pl.BlockSpec((2, r, d), lambda i: (0, i, 0)),
                pl.BlockSpec((r, 8), lambda i: (i, 0)),
                pl.BlockSpec((d, dn), lambda i: (0, 0)),
                pl.BlockSpec((1, dn), lambda i: (0, 0))],
      out_specs=pl.BlockSpec((r, dn), lambda i: (i, 0)),
      out_shape=jax.ShapeDtypeStruct((n, dn), jnp.float32),
  )(tp, dinv8, w, b2d)


def _tc_final(qs, cp, w4, b4_2d, wl, bl2d, g):
  mp, d = qs.shape[1], qs.shape[2]
  mpc = cp.shape[1]
  d4 = w4.shape[1]
  dn = wl.shape[1]

  def body(qs_ref, cp_ref, w4_ref, b4_ref, wl_ref, bl_ref, o_ref):
    qsum = qs_ref[0, :g, :] + qs_ref[1, :g, :]
    cnt = cp_ref[0, :g, 0:1] + cp_ref[1, :g, 0:1]
    maxc = jnp.maximum(cnt, 1.0)
    pooled = jnp.dot(qsum / maxc, w4_ref[...],
                     preferred_element_type=jnp.float32)
    pooled = pooled + (cnt / maxc) * b4_ref[...]
    o_ref[...] = jnp.dot(pooled, wl_ref[...],
                         preferred_element_type=jnp.float32) + bl_ref[...]

  return pl.pallas_call(
      body,
      in_specs=[pl.BlockSpec((2, mp, d), lambda: (0, 0, 0)),
                pl.BlockSpec((2, mpc, 16), lambda: (0, 0, 0)),
                pl.BlockSpec((d, d4), lambda: (0, 0)),
                pl.BlockSpec((1, d4), lambda: (0, 0)),
                pl.BlockSpec((d4, dn), lambda: (0, 0)),
                pl.BlockSpec((1, dn), lambda: (0, 0))],
      out_specs=pl.BlockSpec((g, dn), lambda: (0, 0)),
      out_shape=jax.ShapeDtypeStruct((g, dn), jnp.float32),
  )(qs, cp, w4, b4_2d, wl, bl2d)


def kernel(x, edge_index, batch, W1, b1, W2, b2, W3, b3, W4, b4, Wl, bl):
  n = x.shape[0]
  e = edge_index.shape[1]
  g = 64

  ce = -(-e // (NW * EK))
  ce = ce + (ce % 2)
  e_pad = ce * NW * EK
  src_p = jnp.concatenate(
      [edge_index[0], jnp.zeros((e_pad - e,), jnp.int32)]).reshape(
          NW, ce, EK)
  dst_p = jnp.concatenate(
      [edge_index[1], jnp.full((e_pad - e,), n, jnp.int32)]).reshape(
          NW, ce, EK)
  m = -(-(n + 1) // (NS * 64)) * (NS * 64)
  rt = m // NS

  cpool = -(-n // (NW * EK))
  n_pad = cpool * NW * EK
  bat_p = jnp.concatenate(
      [batch, jnp.full((n_pad - n,), g, jnp.int32)]).reshape(NW, cpool, EK)
  mp = -(-(g + 1) // (NS * 8)) * (NS * 8)
  rtp = mp // NS

  zk = rt // 8
  bat_z = jnp.concatenate(
      [batch, jnp.full((m - n,), g, jnp.int32)]).reshape(NS, rt // zk, zk)

  ones16 = jnp.ones((EK, 16), jnp.float32)
  z16 = jnp.zeros((rt, 16), jnp.float32)
  zc16 = jnp.zeros((rtp, 16), jnp.float32)
  zeros_d = {dd: jnp.zeros((rt, dd), jnp.float32) for dd in (16, 32, 64)}
  zp64 = jnp.zeros((rtp, 64), jnp.float32)

  degp, cntp = _hist_kernel(ce, m, cpool, mp)(
      dst_p, bat_p, ones16, z16, zc16)
  h1 = _tc_matmul(x, W1)
  dinv8, dinv16, g1 = _tc_dinv_g(degp, h1)
  dinv16m = jnp.concatenate(
      [dinv16, jnp.zeros((m - n, 16), jnp.float32)])

  tp1 = _edge_scatter(ce, m, 16, n)(g1, src_p, dst_p, zeros_d[16])
  g2 = _tc_layer1(tp1, dinv8, b1[None, :])
  tp2 = _edge_scatter(ce, m, 16, n)(g2, src_p, dst_p, zeros_d[16])
  g3 = _tc_layer(tp2, dinv8, W2, b2[None, :])
  tp3 = _edge_scatter(ce, m, 32, n)(g3, src_p, dst_p, zeros_d[32])
  g4 = _tc_layer(tp3, dinv8, W3, b3[None, :])

  qs = _edge_pool_scatter(ce, m, 64, n, mp, zk)(
      g4, src_p, dst_p, zeros_d[64], dinv16m, bat_z, zp64)
  return _tc_final(qs, cntp, W4, b4[None, :], Wl, bl[None, :], g)
